# Initial kernel scaffold; baseline (speedup 1.0000x reference)
import jax
import jax.numpy as jnp
from jax.experimental import pallas as pl


def kernel(x, weight):
    raise NotImplementedError("write your pallas kernel here")


# speedup vs baseline: 1.5183x; 1.5183x over previous
import functools

import jax
import jax.numpy as jnp
from jax import lax
from jax.experimental import pallas as pl
from jax.experimental.pallas import tpu as pltpu

NUM_EXPERTS = 64
TOP_K = 2
HIDDEN = 2048
TOKENS = 16384

BT = 512


def _router_block(x_ref, w_ref, rw_ref, ri_ref):
    logits = lax.dot_general(
        x_ref[...], w_ref[...],
        dimension_numbers=(((1,), (1,)), ((), ())),
        preferred_element_type=jnp.float32,
        precision=lax.Precision.DEFAULT,
    )
    iota = lax.broadcasted_iota(jnp.int32, logits.shape, 1)
    m1 = jnp.max(logits, axis=-1, keepdims=True)
    u = jnp.exp(logits - m1)
    p = u / jnp.sum(u, axis=-1, keepdims=True)
    w1 = jnp.max(p, axis=-1, keepdims=True)
    i1 = jnp.min(jnp.where(p == w1, iota, NUM_EXPERTS), axis=-1, keepdims=True)
    masked = jnp.where(iota == i1, -1.0, p)
    w2 = jnp.max(masked, axis=-1, keepdims=True)
    i2 = jnp.min(jnp.where(masked == w2, iota, NUM_EXPERTS), axis=-1, keepdims=True)
    rw_ref[...] = jnp.concatenate([w1, w2], axis=1)
    ri_ref[...] = jnp.concatenate([i1, i2], axis=1)


@functools.partial(jax.jit, static_argnames=("interpret",))
def kernel(x, weight, interpret=False):
    grid = (TOKENS // BT,)
    rw, ri = pl.pallas_call(
        _router_block,
        grid=grid,
        in_specs=[
            pl.BlockSpec((BT, HIDDEN), lambda i: (i, 0)),
            pl.BlockSpec((NUM_EXPERTS, HIDDEN), lambda i: (0, 0)),
        ],
        out_specs=[
            pl.BlockSpec((BT, TOP_K), lambda i: (i, 0)),
            pl.BlockSpec((BT, TOP_K), lambda i: (i, 0)),
        ],
        out_shape=[
            jax.ShapeDtypeStruct((TOKENS, TOP_K), jnp.float32),
            jax.ShapeDtypeStruct((TOKENS, TOP_K), jnp.int32),
        ],
        interpret=interpret,
    )(x, weight)
    return rw, ri


# speedup vs baseline: 1.8037x; 1.1880x over previous
import functools

import jax
import jax.numpy as jnp
from jax import lax
from jax.experimental import pallas as pl
from jax.experimental.pallas import tpu as pltpu

NUM_EXPERTS = 64
TOP_K = 2
HIDDEN = 2048
TOKENS = 16384

BT = 1024


def _router_block(x_ref, w_ref, rw_ref, ri_ref):
    logits = lax.dot_general(
        x_ref[...], w_ref[...],
        dimension_numbers=(((1,), (1,)), ((), ())),
        preferred_element_type=jnp.float32,
        precision=lax.Precision.DEFAULT,
    )
    iota = lax.broadcasted_iota(jnp.int32, logits.shape, 1)
    m1 = jnp.max(logits, axis=-1, keepdims=True)
    u = jnp.exp(logits - m1)
    p = u / jnp.sum(u, axis=-1, keepdims=True)
    w1 = jnp.max(p, axis=-1, keepdims=True)
    i1 = jnp.min(jnp.where(p == w1, iota, NUM_EXPERTS), axis=-1, keepdims=True)
    masked = jnp.where(iota == i1, -1.0, p)
    w2 = jnp.max(masked, axis=-1, keepdims=True)
    i2 = jnp.min(jnp.where(masked == w2, iota, NUM_EXPERTS), axis=-1, keepdims=True)
    rw_ref[...] = jnp.concatenate([w1, w2], axis=1)
    ri_ref[...] = jnp.concatenate([i1, i2], axis=1)


@functools.partial(jax.jit, static_argnames=("interpret",))
def kernel(x, weight, interpret=False):
    grid = (TOKENS // BT,)
    rw, ri = pl.pallas_call(
        _router_block,
        grid=grid,
        in_specs=[
            pl.BlockSpec((BT, HIDDEN), lambda i: (i, 0)),
            pl.BlockSpec((NUM_EXPERTS, HIDDEN), lambda i: (0, 0)),
        ],
        out_specs=[
            pl.BlockSpec((BT, TOP_K), lambda i: (i, 0)),
            pl.BlockSpec((BT, TOP_K), lambda i: (i, 0)),
        ],
        out_shape=[
            jax.ShapeDtypeStruct((TOKENS, TOP_K), jnp.float32),
            jax.ShapeDtypeStruct((TOKENS, TOP_K), jnp.int32),
        ],
        interpret=interpret,
    )(x, weight)
    return rw, ri


# speedup vs baseline: 1.8935x; 1.0498x over previous
import functools

import jax
import jax.numpy as jnp
from jax import lax
from jax.experimental import pallas as pl
from jax.experimental.pallas import tpu as pltpu

NUM_EXPERTS = 64
TOP_K = 2
HIDDEN = 2048
TOKENS = 16384

BT = 2048


def _router_block(x_ref, w_ref, rw_ref, ri_ref):
    logits = lax.dot_general(
        x_ref[...], w_ref[...],
        dimension_numbers=(((1,), (1,)), ((), ())),
        preferred_element_type=jnp.float32,
        precision=lax.Precision.DEFAULT,
    )
    iota = lax.broadcasted_iota(jnp.int32, logits.shape, 1)
    m1 = jnp.max(logits, axis=-1, keepdims=True)
    u = jnp.exp(logits - m1)
    p = u / jnp.sum(u, axis=-1, keepdims=True)
    w1 = jnp.max(p, axis=-1, keepdims=True)
    i1 = jnp.min(jnp.where(p == w1, iota, NUM_EXPERTS), axis=-1, keepdims=True)
    masked = jnp.where(iota == i1, -1.0, p)
    w2 = jnp.max(masked, axis=-1, keepdims=True)
    i2 = jnp.min(jnp.where(masked == w2, iota, NUM_EXPERTS), axis=-1, keepdims=True)
    rw_ref[...] = jnp.concatenate([w1, w2], axis=1)
    ri_ref[...] = jnp.concatenate([i1, i2], axis=1)


@functools.partial(jax.jit, static_argnames=("interpret",))
def kernel(x, weight, interpret=False):
    grid = (TOKENS // BT,)
    rw, ri = pl.pallas_call(
        _router_block,
        grid=grid,
        in_specs=[
            pl.BlockSpec((BT, HIDDEN), lambda i: (i, 0)),
            pl.BlockSpec((NUM_EXPERTS, HIDDEN), lambda i: (0, 0)),
        ],
        out_specs=[
            pl.BlockSpec((BT, TOP_K), lambda i: (i, 0)),
            pl.BlockSpec((BT, TOP_K), lambda i: (i, 0)),
        ],
        out_shape=[
            jax.ShapeDtypeStruct((TOKENS, TOP_K), jnp.float32),
            jax.ShapeDtypeStruct((TOKENS, TOP_K), jnp.int32),
        ],
        interpret=interpret,
    )(x, weight)
    return rw, ri
